# baseline (speedup 1.0000x reference)
import functools

import jax
import jax.numpy as jnp
from jax import lax
from jax.experimental import pallas as pl


def _head_body(x_ref, w1_ref, b1_ref, g1_ref, be1_ref, w2_ref, b2_ref, o_ref):
    x = x_ref[...]
    h = lax.dot_general(x, w1_ref[...], (((1,), (1,)), ((), ())),
                        preferred_element_type=jnp.float32) + b1_ref[...]
    mean = jnp.mean(h, axis=0, keepdims=True)
    var = jnp.mean((h - mean) ** 2, axis=0, keepdims=True)
    h = (h - mean) / jnp.sqrt(var + 1e-5) * g1_ref[...] + be1_ref[...]
    h = jnp.maximum(h, 0.0)
    o_ref[...] = lax.dot_general(h, w2_ref[...], (((1,), (1,)), ((), ())),
                                 preferred_element_type=jnp.float32) + b2_ref[...]


def _head(x, w1, b1, g1, be1, w2, b2):
    B, N, C = x.shape
    out = pl.pallas_call(
        _head_body,
        out_shape=jax.ShapeDtypeStruct((B * N, w2.shape[0]), jnp.float32),
    )(x.reshape(B * N, C), w1, b1.reshape(1, -1), g1.reshape(1, -1),
      be1.reshape(1, -1), w2, b2.reshape(1, -1))
    return out.reshape(B, N, w2.shape[0])


def _square_distance(src, dst):
    return (jnp.sum(src ** 2, -1)[:, :, None]
            - 2.0 * jnp.einsum('bnc,bmc->bnm', src, dst)
            + jnp.sum(dst ** 2, -1)[:, None, :])


def _index_points(points, idx):
    B = points.shape[0]
    batch = jnp.arange(B).reshape((B,) + (1,) * (idx.ndim - 1))
    return points[batch, idx]


def _farthest_point_sample(xyz, npoint):
    B, N, _ = xyz.shape
    def body(i, state):
        centroids, distance, farthest = state
        centroids = centroids.at[:, i].set(farthest)
        centroid = jnp.take_along_axis(xyz, farthest[:, None, None], axis=1)
        dist = jnp.sum((xyz - centroid) ** 2, -1)
        distance = jnp.minimum(distance, dist)
        farthest = jnp.argmax(distance, -1).astype(jnp.int32)
        return centroids, distance, farthest
    centroids = jnp.zeros((B, npoint), dtype=jnp.int32)
    distance = jnp.full((B, N), 1e10, dtype=xyz.dtype)
    farthest = jnp.zeros((B,), dtype=jnp.int32)
    centroids, _, _ = jax.lax.fori_loop(0, npoint, body,
                                        (centroids, distance, farthest))
    return centroids


def _query_ball_point(radius, nsample, xyz, new_xyz):
    B, N, _ = xyz.shape
    S = new_xyz.shape[1]
    sqrdists = _square_distance(new_xyz, xyz)
    group_idx = jnp.broadcast_to(jnp.arange(N, dtype=jnp.int32), (B, S, N))
    group_idx = jnp.where(sqrdists > radius ** 2, N, group_idx)
    group_idx = jnp.sort(group_idx, axis=-1)[:, :, :nsample]
    group_first = group_idx[:, :, :1]
    group_idx = jnp.where(group_idx == N, group_first, group_idx)
    return group_idx


def _batchnorm(x, g, b, axes):
    mean = jnp.mean(x, axis=axes, keepdims=True)
    var = jnp.var(x, axis=axes, keepdims=True)
    return (x - mean) / jnp.sqrt(var + 1e-5) * g + b


def _mlp(x, p, prefix, nlayers, axes):
    for i in range(nlayers):
        w = p[prefix + '_w%d' % i]
        b = p[prefix + '_b%d' % i]
        x = jnp.einsum('...c,oc->...o', x, w) + b
        x = _batchnorm(x, p[prefix + '_g%d' % i], p[prefix + '_be%d' % i], axes)
        x = jax.nn.relu(x)
    return x


def _set_abstraction(p, prefix, nlayers, xyz, points, npoint, radius, nsample,
                     group_all):
    B, N, _ = xyz.shape
    if group_all:
        new_xyz = jnp.zeros((B, 1, 3), dtype=xyz.dtype)
        grouped = xyz[:, None, :, :]
        if points is not None:
            grouped = jnp.concatenate([grouped, points[:, None, :, :]], -1)
    else:
        fps_idx = _farthest_point_sample(xyz, npoint)
        new_xyz = _index_points(xyz, fps_idx)
        idx = _query_ball_point(radius, nsample, xyz, new_xyz)
        grouped = _index_points(xyz, idx) - new_xyz[:, :, None, :]
        if points is not None:
            grouped = jnp.concatenate([grouped, _index_points(points, idx)], -1)
    new_points = _mlp(grouped, p, prefix, nlayers, (0, 1, 2))
    new_points = jnp.max(new_points, axis=2)
    return new_xyz, new_points


def _feature_propagation(p, prefix, nlayers, xyz1, xyz2, points1, points2):
    B, N, _ = xyz1.shape
    S = xyz2.shape[1]
    if S == 1:
        interpolated = jnp.broadcast_to(points2, (B, N, points2.shape[-1]))
    else:
        dists = _square_distance(xyz1, xyz2)
        idx = jnp.argsort(dists, axis=-1)[:, :, :3]
        d = jnp.take_along_axis(dists, idx, axis=-1)
        dist_recip = 1.0 / (d + 1e-8)
        norm = jnp.sum(dist_recip, axis=-1, keepdims=True)
        weight = dist_recip / norm
        interpolated = jnp.sum(_index_points(points2, idx) * weight[..., None],
                               axis=2)
    if points1 is not None:
        new_points = jnp.concatenate([points1, interpolated], -1)
    else:
        new_points = interpolated
    return _mlp(new_points, p, prefix, nlayers, (0, 1))


def kernel(xyz, sa1_w0, sa1_b0, sa1_g0, sa1_be0, sa1_w1, sa1_b1, sa1_g1, sa1_be1, sa1_w2, sa1_b2, sa1_g2, sa1_be2, sa2_w0, sa2_b0, sa2_g0, sa2_be0, sa2_w1, sa2_b1, sa2_g1, sa2_be1, sa2_w2, sa2_b2, sa2_g2, sa2_be2, sa3_w0, sa3_b0, sa3_g0, sa3_be0, sa3_w1, sa3_b1, sa3_g1, sa3_be1, sa3_w2, sa3_b2, sa3_g2, sa3_be2, fp3_w0, fp3_b0, fp3_g0, fp3_be0, fp3_w1, fp3_b1, fp3_g1, fp3_be1, fp2_w0, fp2_b0, fp2_g0, fp2_be0, fp2_w1, fp2_b1, fp2_g1, fp2_be1, fp1_w0, fp1_b0, fp1_g0, fp1_be0, fp1_w1, fp1_b1, fp1_g1, fp1_be1, fp1_w2, fp1_b2, fp1_g2, fp1_be2, head_w1, head_b1, head_g1, head_be1, head_w2, head_b2):
    p = dict(locals())
    x0 = jnp.transpose(p['xyz'], (0, 2, 1))
    l1_xyz, l1_points = _set_abstraction(p, 'sa1', 3, x0, None, 512, 0.1, 64, False)
    l2_xyz, l2_points = _set_abstraction(p, 'sa2', 3, l1_xyz, l1_points, 128, 0.2, 64, False)
    l3_xyz, l3_points = _set_abstraction(p, 'sa3', 3, l2_xyz, l2_points, None, None, None, True)
    l2_points = _feature_propagation(p, 'fp3', 2, l2_xyz, l3_xyz, l2_points, l3_points)
    l1_points = _feature_propagation(p, 'fp2', 2, l1_xyz, l2_xyz, l1_points, l2_points)
    l0_points = _feature_propagation(p, 'fp1', 3, x0, l1_xyz, None, l1_points)
    x = _head(l0_points, head_w1, head_b1, head_g1, head_be1, head_w2, head_b2)
    return jnp.transpose(x, (0, 2, 1))


# speedup vs baseline: 1.3488x; 1.3488x over previous
import functools

import jax
import jax.numpy as jnp
from jax import lax
from jax.experimental import pallas as pl


def _head_body(x_ref, w1_ref, b1_ref, g1_ref, be1_ref, w2_ref, b2_ref, o_ref):
    x = x_ref[...]
    h = lax.dot_general(x, w1_ref[...], (((1,), (1,)), ((), ())),
                        preferred_element_type=jnp.float32) + b1_ref[...]
    mean = jnp.mean(h, axis=0, keepdims=True)
    var = jnp.mean((h - mean) ** 2, axis=0, keepdims=True)
    h = (h - mean) / jnp.sqrt(var + 1e-5) * g1_ref[...] + be1_ref[...]
    h = jnp.maximum(h, 0.0)
    o_ref[...] = lax.dot_general(h, w2_ref[...], (((1,), (1,)), ((), ())),
                                 preferred_element_type=jnp.float32) + b2_ref[...]


def _head(x, w1, b1, g1, be1, w2, b2):
    B, N, C = x.shape
    out = pl.pallas_call(
        _head_body,
        out_shape=jax.ShapeDtypeStruct((B * N, w2.shape[0]), jnp.float32),
    )(x.reshape(B * N, C), w1, b1.reshape(1, -1), g1.reshape(1, -1),
      be1.reshape(1, -1), w2, b2.reshape(1, -1))
    return out.reshape(B, N, w2.shape[0])


def _fps_body(xyz_ref, cent_ref, nxyz_ref, *, npoint):
    x = xyz_ref[:, 0, :]
    y = xyz_ref[:, 1, :]
    z = xyz_ref[:, 2, :]
    B, N = x.shape
    iota = lax.broadcasted_iota(jnp.int32, (B, N), 1)
    iota_np = lax.broadcasted_iota(jnp.int32, (B, npoint), 1)

    cent_ref[...] = jnp.zeros_like(cent_ref)
    nxyz_ref[...] = jnp.zeros_like(nxyz_ref)

    def body(i, dist):
        amax = jnp.argmax(dist, axis=1, keepdims=True).astype(jnp.int32)
        far = jnp.where(i == 0, jnp.zeros_like(amax), amax)
        seli = (iota_np == i).astype(jnp.int32)
        self_ = seli.astype(jnp.float32)
        cent_ref[...] = cent_ref[...] + seli * (far - cent_ref[...])
        oh = (iota == far).astype(jnp.float32)
        cx = jnp.sum(x * oh, axis=1, keepdims=True)
        cy = jnp.sum(y * oh, axis=1, keepdims=True)
        cz = jnp.sum(z * oh, axis=1, keepdims=True)
        nxyz_ref[:, 0, :] += self_ * (cx - nxyz_ref[:, 0, :])
        nxyz_ref[:, 1, :] += self_ * (cy - nxyz_ref[:, 1, :])
        nxyz_ref[:, 2, :] += self_ * (cz - nxyz_ref[:, 2, :])
        d = (x - cx) ** 2 + (y - cy) ** 2 + (z - cz) ** 2
        return jnp.minimum(dist, d)

    dist0 = jnp.full((B, N), 1e10, dtype=jnp.float32)
    lax.fori_loop(0, npoint, body, dist0)


def _fps(xyz_b3n, npoint):
    B, _, N = xyz_b3n.shape
    cent, nxyz = pl.pallas_call(
        functools.partial(_fps_body, npoint=npoint),
        out_shape=(jax.ShapeDtypeStruct((B, npoint), jnp.int32),
                   jax.ShapeDtypeStruct((B, 3, npoint), jnp.float32)),
    )(xyz_b3n)
    return cent, nxyz


def _square_distance(src, dst):
    return (jnp.sum(src ** 2, -1)[:, :, None]
            - 2.0 * jnp.einsum('bnc,bmc->bnm', src, dst)
            + jnp.sum(dst ** 2, -1)[:, None, :])


def _index_points(points, idx):
    B = points.shape[0]
    batch = jnp.arange(B).reshape((B,) + (1,) * (idx.ndim - 1))
    return points[batch, idx]


def _farthest_point_sample(xyz, npoint):
    B, N, _ = xyz.shape
    def body(i, state):
        centroids, distance, farthest = state
        centroids = centroids.at[:, i].set(farthest)
        centroid = jnp.take_along_axis(xyz, farthest[:, None, None], axis=1)
        dist = jnp.sum((xyz - centroid) ** 2, -1)
        distance = jnp.minimum(distance, dist)
        farthest = jnp.argmax(distance, -1).astype(jnp.int32)
        return centroids, distance, farthest
    centroids = jnp.zeros((B, npoint), dtype=jnp.int32)
    distance = jnp.full((B, N), 1e10, dtype=xyz.dtype)
    farthest = jnp.zeros((B,), dtype=jnp.int32)
    centroids, _, _ = jax.lax.fori_loop(0, npoint, body,
                                        (centroids, distance, farthest))
    return centroids


def _query_ball_point(radius, nsample, xyz, new_xyz):
    B, N, _ = xyz.shape
    S = new_xyz.shape[1]
    sqrdists = _square_distance(new_xyz, xyz)
    group_idx = jnp.broadcast_to(jnp.arange(N, dtype=jnp.int32), (B, S, N))
    group_idx = jnp.where(sqrdists > radius ** 2, N, group_idx)
    group_idx = jnp.sort(group_idx, axis=-1)[:, :, :nsample]
    group_first = group_idx[:, :, :1]
    group_idx = jnp.where(group_idx == N, group_first, group_idx)
    return group_idx


def _batchnorm(x, g, b, axes):
    mean = jnp.mean(x, axis=axes, keepdims=True)
    var = jnp.var(x, axis=axes, keepdims=True)
    return (x - mean) / jnp.sqrt(var + 1e-5) * g + b


def _mlp(x, p, prefix, nlayers, axes):
    for i in range(nlayers):
        w = p[prefix + '_w%d' % i]
        b = p[prefix + '_b%d' % i]
        x = jnp.einsum('...c,oc->...o', x, w) + b
        x = _batchnorm(x, p[prefix + '_g%d' % i], p[prefix + '_be%d' % i], axes)
        x = jax.nn.relu(x)
    return x


def _set_abstraction(p, prefix, nlayers, xyz, points, npoint, radius, nsample,
                     group_all):
    B, N, _ = xyz.shape
    if group_all:
        new_xyz = jnp.zeros((B, 1, 3), dtype=xyz.dtype)
        grouped = xyz[:, None, :, :]
        if points is not None:
            grouped = jnp.concatenate([grouped, points[:, None, :, :]], -1)
    else:
        _, nxyz = _fps(jnp.transpose(xyz, (0, 2, 1)), npoint)
        new_xyz = jnp.transpose(nxyz, (0, 2, 1))
        idx = _query_ball_point(radius, nsample, xyz, new_xyz)
        grouped = _index_points(xyz, idx) - new_xyz[:, :, None, :]
        if points is not None:
            grouped = jnp.concatenate([grouped, _index_points(points, idx)], -1)
    new_points = _mlp(grouped, p, prefix, nlayers, (0, 1, 2))
    new_points = jnp.max(new_points, axis=2)
    return new_xyz, new_points


def _feature_propagation(p, prefix, nlayers, xyz1, xyz2, points1, points2):
    B, N, _ = xyz1.shape
    S = xyz2.shape[1]
    if S == 1:
        interpolated = jnp.broadcast_to(points2, (B, N, points2.shape[-1]))
    else:
        dists = _square_distance(xyz1, xyz2)
        idx = jnp.argsort(dists, axis=-1)[:, :, :3]
        d = jnp.take_along_axis(dists, idx, axis=-1)
        dist_recip = 1.0 / (d + 1e-8)
        norm = jnp.sum(dist_recip, axis=-1, keepdims=True)
        weight = dist_recip / norm
        interpolated = jnp.sum(_index_points(points2, idx) * weight[..., None],
                               axis=2)
    if points1 is not None:
        new_points = jnp.concatenate([points1, interpolated], -1)
    else:
        new_points = interpolated
    return _mlp(new_points, p, prefix, nlayers, (0, 1))


def kernel(xyz, sa1_w0, sa1_b0, sa1_g0, sa1_be0, sa1_w1, sa1_b1, sa1_g1, sa1_be1, sa1_w2, sa1_b2, sa1_g2, sa1_be2, sa2_w0, sa2_b0, sa2_g0, sa2_be0, sa2_w1, sa2_b1, sa2_g1, sa2_be1, sa2_w2, sa2_b2, sa2_g2, sa2_be2, sa3_w0, sa3_b0, sa3_g0, sa3_be0, sa3_w1, sa3_b1, sa3_g1, sa3_be1, sa3_w2, sa3_b2, sa3_g2, sa3_be2, fp3_w0, fp3_b0, fp3_g0, fp3_be0, fp3_w1, fp3_b1, fp3_g1, fp3_be1, fp2_w0, fp2_b0, fp2_g0, fp2_be0, fp2_w1, fp2_b1, fp2_g1, fp2_be1, fp1_w0, fp1_b0, fp1_g0, fp1_be0, fp1_w1, fp1_b1, fp1_g1, fp1_be1, fp1_w2, fp1_b2, fp1_g2, fp1_be2, head_w1, head_b1, head_g1, head_be1, head_w2, head_b2):
    p = dict(locals())
    x0 = jnp.transpose(p['xyz'], (0, 2, 1))
    l1_xyz, l1_points = _set_abstraction(p, 'sa1', 3, x0, None, 512, 0.1, 64, False)
    l2_xyz, l2_points = _set_abstraction(p, 'sa2', 3, l1_xyz, l1_points, 128, 0.2, 64, False)
    l3_xyz, l3_points = _set_abstraction(p, 'sa3', 3, l2_xyz, l2_points, None, None, None, True)
    l2_points = _feature_propagation(p, 'fp3', 2, l2_xyz, l3_xyz, l2_points, l3_points)
    l1_points = _feature_propagation(p, 'fp2', 2, l1_xyz, l2_xyz, l1_points, l2_points)
    l0_points = _feature_propagation(p, 'fp1', 3, x0, l1_xyz, None, l1_points)
    x = _head(l0_points, head_w1, head_b1, head_g1, head_be1, head_w2, head_b2)
    return jnp.transpose(x, (0, 2, 1))


# speedup vs baseline: 1.9537x; 1.4484x over previous
import functools

import jax
import jax.numpy as jnp
from jax import lax
from jax.experimental import pallas as pl
from jax.experimental.pallas import tpu as pltpu


def _head_body(x_ref, w1_ref, b1_ref, g1_ref, be1_ref, w2_ref, b2_ref, o_ref):
    x = x_ref[...]
    h = lax.dot_general(x, w1_ref[...], (((1,), (1,)), ((), ())),
                        preferred_element_type=jnp.float32) + b1_ref[...]
    mean = jnp.mean(h, axis=0, keepdims=True)
    var = jnp.mean((h - mean) ** 2, axis=0, keepdims=True)
    h = (h - mean) / jnp.sqrt(var + 1e-5) * g1_ref[...] + be1_ref[...]
    h = jnp.maximum(h, 0.0)
    o_ref[...] = lax.dot_general(h, w2_ref[...], (((1,), (1,)), ((), ())),
                                 preferred_element_type=jnp.float32) + b2_ref[...]


def _head(x, w1, b1, g1, be1, w2, b2):
    B, N, C = x.shape
    out = pl.pallas_call(
        _head_body,
        out_shape=jax.ShapeDtypeStruct((B * N, w2.shape[0]), jnp.float32),
    )(x.reshape(B * N, C), w1, b1.reshape(1, -1), g1.reshape(1, -1),
      be1.reshape(1, -1), w2, b2.reshape(1, -1))
    return out.reshape(B, N, w2.shape[0])


def _fps_body(xyz_ref, cent_ref, nxyz_ref, *, npoint):
    x = xyz_ref[:, 0, :]
    y = xyz_ref[:, 1, :]
    z = xyz_ref[:, 2, :]
    B, N = x.shape
    iota = lax.broadcasted_iota(jnp.int32, (B, N), 1)
    iota_np = lax.broadcasted_iota(jnp.int32, (B, npoint), 1)

    cent_ref[...] = jnp.zeros_like(cent_ref)
    nxyz_ref[...] = jnp.zeros_like(nxyz_ref)

    def body(i, dist):
        amax = jnp.argmax(dist, axis=1, keepdims=True).astype(jnp.int32)
        far = jnp.where(i == 0, jnp.zeros_like(amax), amax)
        seli = (iota_np == i).astype(jnp.int32)
        self_ = seli.astype(jnp.float32)
        cent_ref[...] = cent_ref[...] + seli * (far - cent_ref[...])
        oh = (iota == far).astype(jnp.float32)
        cx = jnp.sum(x * oh, axis=1, keepdims=True)
        cy = jnp.sum(y * oh, axis=1, keepdims=True)
        cz = jnp.sum(z * oh, axis=1, keepdims=True)
        nxyz_ref[:, 0, :] += self_ * (cx - nxyz_ref[:, 0, :])
        nxyz_ref[:, 1, :] += self_ * (cy - nxyz_ref[:, 1, :])
        nxyz_ref[:, 2, :] += self_ * (cz - nxyz_ref[:, 2, :])
        d = (x - cx) ** 2 + (y - cy) ** 2 + (z - cz) ** 2
        return jnp.minimum(dist, d)

    dist0 = jnp.full((B, N), 1e10, dtype=jnp.float32)
    lax.fori_loop(0, npoint, body, dist0)


def _fps(xyz_b3n, npoint):
    B, _, N = xyz_b3n.shape
    cent, nxyz = pl.pallas_call(
        functools.partial(_fps_body, npoint=npoint),
        out_shape=(jax.ShapeDtypeStruct((B, npoint), jnp.int32),
                   jax.ShapeDtypeStruct((B, 3, npoint), jnp.float32)),
    )(xyz_b3n)
    return cent, nxyz


def _gather_wide(arr, idx):
    TS, M = arr.shape
    if M <= 128:
        return jnp.take_along_axis(arr, jnp.clip(idx, 0, M - 1), axis=1)
    out = None
    for c in range(M // 128):
        part = jnp.take_along_axis(arr[:, c * 128:(c + 1) * 128],
                                   jnp.clip(idx - c * 128, 0, 127), axis=1)
        out = part if out is None else jnp.where(idx >= c * 128, part, out)
    return out


def _popcount16(v):
    v = v - ((v >> 1) & 0x5555)
    v = (v & 0x3333) + ((v >> 2) & 0x3333)
    v = (v + (v >> 4)) & 0x0F0F
    return (v + (v >> 8)) & 0x1F


def _bq_body(xyz_ref, nxyz_ref, gmat_ref, out_ref, *, r2, nsample):
    xs = xyz_ref[0]
    s = nxyz_ref[0]
    N = xs.shape[1]
    TS = s.shape[0]
    NG = N // 16
    px2 = jnp.sum(xs * xs, axis=0, keepdims=True)
    ns2 = jnp.sum(s * s, axis=1, keepdims=True)
    dot = lax.dot_general(s, xs, (((1,), (0,)), ((), ())),
                          preferred_element_type=jnp.float32)
    d = (ns2 - 2.0 * dot) + px2
    maskf = (d <= r2).astype(jnp.float32)

    bits = lax.dot_general(maskf, gmat_ref[...], (((1,), (0,)), ((), ())),
                           preferred_element_type=jnp.float32)
    bits = bits.astype(jnp.int32)
    gc = _popcount16(bits)
    lane = lax.broadcasted_iota(jnp.int32, (TS, NG), 1)
    fp = gc
    sh = 1
    while sh < NG:
        fp = fp + jnp.where(lane >= sh, pltpu.roll(fp, sh, axis=1), 0)
        sh *= 2

    iota_k = lax.broadcasted_iota(jnp.int32, (TS, nsample), 1)
    t = iota_k + 1
    lo = jnp.zeros((TS, nsample), dtype=jnp.int32)
    hi = jnp.full((TS, nsample), NG, dtype=jnp.int32)
    for _ in range(int(NG + 1).bit_length()):
        mid = (lo + hi) // 2
        v = _gather_wide(fp, jnp.minimum(mid, NG - 1))
        cond = v < t
        lo = jnp.where(cond, mid + 1, lo)
        hi = jnp.where(cond, hi, mid)
    g = jnp.minimum(lo, NG - 1)

    prior = jnp.where(g > 0, _gather_wide(fp, g - 1), 0)
    rem = t - prior
    w = _gather_wide(bits, g)
    u = jnp.zeros((TS, nsample), dtype=jnp.int32)
    for h in (8, 4, 2, 1):
        v8 = (w >> u) & ((1 << h) - 1)
        v8 = v8 - ((v8 >> 1) & 0x5555)
        v8 = (v8 & 0x3333) + ((v8 >> 2) & 0x3333)
        v8 = (v8 + (v8 >> 4)) & 0x0F
        c = (v8 < rem).astype(jnp.int32)
        u = u + h * c
        rem = rem - v8 * c
    idx = g * 16 + u

    count = fp[:, NG - 1:NG]
    first = idx[:, 0:1]
    out_ref[0] = jnp.where(t <= count, idx, first)


def _ball_query(xyz_b3n, new_xyz, radius, nsample, ts):
    B, _, N = xyz_b3n.shape
    S = new_xyz.shape[1]
    NG = N // 16
    n_iota = jnp.arange(N, dtype=jnp.int32)
    gmat = jnp.where((n_iota[:, None] // 16) == jnp.arange(NG)[None, :],
                     (2.0 ** (n_iota % 16))[:, None], 0.0).astype(jnp.float32)
    return pl.pallas_call(
        functools.partial(_bq_body, r2=radius * radius, nsample=nsample),
        grid=(B, S // ts),
        in_specs=[
            pl.BlockSpec((1, 3, N), lambda b, t: (b, 0, 0)),
            pl.BlockSpec((1, ts, 3), lambda b, t: (b, t, 0)),
            pl.BlockSpec((N, NG), lambda b, t: (0, 0)),
        ],
        out_specs=pl.BlockSpec((1, ts, nsample), lambda b, t: (b, t, 0)),
        out_shape=jax.ShapeDtypeStruct((B, S, nsample), jnp.int32),
    )(xyz_b3n, new_xyz, gmat)


def _nn3_body(xyz1_ref, xyz2_ref, idx_ref, w_ref):
    s = xyz1_ref[0]
    xs = xyz2_ref[0]
    TS = s.shape[0]
    N2 = xs.shape[1]
    px2 = jnp.sum(xs * xs, axis=0, keepdims=True)
    ns2 = jnp.sum(s * s, axis=1, keepdims=True)
    dot = lax.dot_general(s, xs, (((1,), (0,)), ((), ())),
                          preferred_element_type=jnp.float32)
    d = (ns2 - 2.0 * dot) + px2
    lane = lax.broadcasted_iota(jnp.int32, (TS, N2), 1)
    idxs, ds = [], []
    for _ in range(3):
        am = jnp.argmin(d, axis=1, keepdims=True).astype(jnp.int32)
        dv = jnp.min(d, axis=1, keepdims=True)
        idxs.append(am)
        ds.append(dv)
        d = jnp.where(lane == am, 1e30, d)
    idx_ref[0] = jnp.concatenate(idxs, axis=1)
    d3 = jnp.concatenate(ds, axis=1)
    recip = 1.0 / (d3 + 1e-8)
    w_ref[0] = recip / jnp.sum(recip, axis=1, keepdims=True)


def _nn3(xyz1, xyz2_b3n, ts):
    B, N1, _ = xyz1.shape
    N2 = xyz2_b3n.shape[2]
    return pl.pallas_call(
        _nn3_body,
        grid=(B, N1 // ts),
        in_specs=[
            pl.BlockSpec((1, ts, 3), lambda b, t: (b, t, 0)),
            pl.BlockSpec((1, 3, N2), lambda b, t: (b, 0, 0)),
        ],
        out_specs=[
            pl.BlockSpec((1, ts, 3), lambda b, t: (b, t, 0)),
            pl.BlockSpec((1, ts, 3), lambda b, t: (b, t, 0)),
        ],
        out_shape=[jax.ShapeDtypeStruct((B, N1, 3), jnp.int32),
                   jax.ShapeDtypeStruct((B, N1, 3), jnp.float32)],
    )(xyz1, xyz2_b3n)


def _square_distance(src, dst):
    return (jnp.sum(src ** 2, -1)[:, :, None]
            - 2.0 * jnp.einsum('bnc,bmc->bnm', src, dst)
            + jnp.sum(dst ** 2, -1)[:, None, :])


def _index_points(points, idx):
    B = points.shape[0]
    batch = jnp.arange(B).reshape((B,) + (1,) * (idx.ndim - 1))
    return points[batch, idx]


def _farthest_point_sample(xyz, npoint):
    B, N, _ = xyz.shape
    def body(i, state):
        centroids, distance, farthest = state
        centroids = centroids.at[:, i].set(farthest)
        centroid = jnp.take_along_axis(xyz, farthest[:, None, None], axis=1)
        dist = jnp.sum((xyz - centroid) ** 2, -1)
        distance = jnp.minimum(distance, dist)
        farthest = jnp.argmax(distance, -1).astype(jnp.int32)
        return centroids, distance, farthest
    centroids = jnp.zeros((B, npoint), dtype=jnp.int32)
    distance = jnp.full((B, N), 1e10, dtype=xyz.dtype)
    farthest = jnp.zeros((B,), dtype=jnp.int32)
    centroids, _, _ = jax.lax.fori_loop(0, npoint, body,
                                        (centroids, distance, farthest))
    return centroids


def _query_ball_point(radius, nsample, xyz, new_xyz):
    B, N, _ = xyz.shape
    S = new_xyz.shape[1]
    sqrdists = _square_distance(new_xyz, xyz)
    group_idx = jnp.broadcast_to(jnp.arange(N, dtype=jnp.int32), (B, S, N))
    group_idx = jnp.where(sqrdists > radius ** 2, N, group_idx)
    group_idx = jnp.sort(group_idx, axis=-1)[:, :, :nsample]
    group_first = group_idx[:, :, :1]
    group_idx = jnp.where(group_idx == N, group_first, group_idx)
    return group_idx


def _batchnorm(x, g, b, axes):
    mean = jnp.mean(x, axis=axes, keepdims=True)
    var = jnp.var(x, axis=axes, keepdims=True)
    return (x - mean) / jnp.sqrt(var + 1e-5) * g + b


def _mlp(x, p, prefix, nlayers, axes):
    for i in range(nlayers):
        w = p[prefix + '_w%d' % i]
        b = p[prefix + '_b%d' % i]
        x = jnp.einsum('...c,oc->...o', x, w) + b
        x = _batchnorm(x, p[prefix + '_g%d' % i], p[prefix + '_be%d' % i], axes)
        x = jax.nn.relu(x)
    return x


def _set_abstraction(p, prefix, nlayers, xyz, points, npoint, radius, nsample,
                     group_all):
    B, N, _ = xyz.shape
    if group_all:
        new_xyz = jnp.zeros((B, 1, 3), dtype=xyz.dtype)
        grouped = xyz[:, None, :, :]
        if points is not None:
            grouped = jnp.concatenate([grouped, points[:, None, :, :]], -1)
    else:
        xyz_b3n = jnp.transpose(xyz, (0, 2, 1))
        _, nxyz = _fps(xyz_b3n, npoint)
        new_xyz = jnp.transpose(nxyz, (0, 2, 1))
        idx = _ball_query(xyz_b3n, new_xyz, radius, nsample,
                          ts=min(npoint, 256))
        grouped = _index_points(xyz, idx) - new_xyz[:, :, None, :]
        if points is not None:
            grouped = jnp.concatenate([grouped, _index_points(points, idx)], -1)
    new_points = _mlp(grouped, p, prefix, nlayers, (0, 1, 2))
    new_points = jnp.max(new_points, axis=2)
    return new_xyz, new_points


def _feature_propagation(p, prefix, nlayers, xyz1, xyz2, points1, points2):
    B, N, _ = xyz1.shape
    S = xyz2.shape[1]
    if S == 1:
        interpolated = jnp.broadcast_to(points2, (B, N, points2.shape[-1]))
    else:
        idx, weight = _nn3(xyz1, jnp.transpose(xyz2, (0, 2, 1)),
                           ts=min(N, 512))
        interpolated = jnp.sum(_index_points(points2, idx) * weight[..., None],
                               axis=2)
    if points1 is not None:
        new_points = jnp.concatenate([points1, interpolated], -1)
    else:
        new_points = interpolated
    return _mlp(new_points, p, prefix, nlayers, (0, 1))


def kernel(xyz, sa1_w0, sa1_b0, sa1_g0, sa1_be0, sa1_w1, sa1_b1, sa1_g1, sa1_be1, sa1_w2, sa1_b2, sa1_g2, sa1_be2, sa2_w0, sa2_b0, sa2_g0, sa2_be0, sa2_w1, sa2_b1, sa2_g1, sa2_be1, sa2_w2, sa2_b2, sa2_g2, sa2_be2, sa3_w0, sa3_b0, sa3_g0, sa3_be0, sa3_w1, sa3_b1, sa3_g1, sa3_be1, sa3_w2, sa3_b2, sa3_g2, sa3_be2, fp3_w0, fp3_b0, fp3_g0, fp3_be0, fp3_w1, fp3_b1, fp3_g1, fp3_be1, fp2_w0, fp2_b0, fp2_g0, fp2_be0, fp2_w1, fp2_b1, fp2_g1, fp2_be1, fp1_w0, fp1_b0, fp1_g0, fp1_be0, fp1_w1, fp1_b1, fp1_g1, fp1_be1, fp1_w2, fp1_b2, fp1_g2, fp1_be2, head_w1, head_b1, head_g1, head_be1, head_w2, head_b2):
    p = dict(locals())
    x0 = jnp.transpose(p['xyz'], (0, 2, 1))
    l1_xyz, l1_points = _set_abstraction(p, 'sa1', 3, x0, None, 512, 0.1, 64, False)
    l2_xyz, l2_points = _set_abstraction(p, 'sa2', 3, l1_xyz, l1_points, 128, 0.2, 64, False)
    l3_xyz, l3_points = _set_abstraction(p, 'sa3', 3, l2_xyz, l2_points, None, None, None, True)
    l2_points = _feature_propagation(p, 'fp3', 2, l2_xyz, l3_xyz, l2_points, l3_points)
    l1_points = _feature_propagation(p, 'fp2', 2, l1_xyz, l2_xyz, l1_points, l2_points)
    l0_points = _feature_propagation(p, 'fp1', 3, x0, l1_xyz, None, l1_points)
    x = _head(l0_points, head_w1, head_b1, head_g1, head_be1, head_w2, head_b2)
    return jnp.transpose(x, (0, 2, 1))
